# Initial kernel scaffold; baseline (speedup 1.0000x reference)
import jax
import jax.numpy as jnp
from jax.experimental import pallas as pl


def kernel(nodes, edges, senders, receivers, We1, be1, We2, be2, We3, be3, Wn1, bn1, Wn2, bn2, Wg1, bg1, Wg2, bg2, Wg3, bg3):
    raise NotImplementedError("write your pallas kernel here")


# speedup vs baseline: 1.0740x; 1.0740x over previous
import functools

import jax
import jax.numpy as jnp
from jax import lax
from jax.experimental import pallas as pl
from jax.experimental.pallas import tpu as pltpu
from jax.experimental.pallas import tpu_sc as plsc

N = 50000
NP = 50048
E = 800000
EP = 819200
EBLK = 1024
NEB = EP // EBLK
IDX_ROWS = EP // 128

NC, NS = 2, 16
NW = NC * NS

G_ROWS = IDX_ROWS // NW
G_BLK = 8
G_NBLK = G_ROWS // G_BLK

S_ROWS = IDX_ROWS // NS
S_BLK = 8
S_NBLK = S_ROWS // S_BLK

CHUNKS = 8
CW = 16
NPC = CHUNKS // NC
ZR = NP // NS
ZB = 136

NB = 1000
NNB = N // NB

_ALPHA = 1.6732632423543772
_SCALE = 1.0507009873554805


def _selu(x):
    return _SCALE * jnp.where(x > 0, x, _ALPHA * (jnp.exp(x) - 1.0))


@functools.cache
def _sc_mesh():
    return plsc.VectorSubcoreMesh(
        core_axis_name="c", subcore_axis_name="s",
        num_cores=NC, num_subcores=NS)


@functools.cache
def _sc_gather_kernel():
    return functools.partial(
        pl.kernel,
        out_type=(jax.ShapeDtypeStruct((EP, 32), jnp.float32),
                  jax.ShapeDtypeStruct((EP, 32), jnp.float32)),
        mesh=_sc_mesh(),
        scratch_types=[
            pltpu.VMEM((G_BLK, 128), jnp.int32),
            pltpu.VMEM((G_BLK * 128, 32), jnp.float32),
            pltpu.SemaphoreType.DMA,
        ],
        compiler_params=pltpu.CompilerParams(use_tc_tiling_on_sc=False),
    )(_sc_gather_body)


def _sc_gather_body(table, send2d, recv2d, out_s, out_r, idx_v, rows_v, sem):
    wid = lax.axis_index("s") * NC + lax.axis_index("c")
    row0 = wid * G_ROWS

    def run(idx2d, out):
        def blk(b, carry):
            r0 = row0 + b * G_BLK
            pltpu.sync_copy(idx2d.at[pl.ds(r0, G_BLK)], idx_v)
            descs = [
                pltpu.async_copy(table.at[idx_v.at[j]],
                                 rows_v.at[pl.ds(j * 128, 128)], sem)
                for j in range(G_BLK)
            ]
            for d in descs:
                d.wait()
            pltpu.sync_copy(rows_v, out.at[pl.ds(r0 * 128, G_BLK * 128)])
            return carry
        lax.fori_loop(0, G_NBLK, blk, 0)

    run(send2d, out_s)
    run(recv2d, out_r)


@functools.cache
def _sc_scatter_kernel():
    return functools.partial(
        pl.kernel,
        out_type=jax.ShapeDtypeStruct((CHUNKS * NP, CW), jnp.float32),
        mesh=_sc_mesh(),
        scratch_types=[
            pltpu.VMEM((S_BLK, 128), jnp.int32),
            pltpu.VMEM((S_BLK * 128, CW), jnp.float32),
            pltpu.VMEM((ZB, CW), jnp.float32),
            pltpu.VMEM_SHARED((NP, CW), jnp.float32),
            pltpu.SemaphoreType.DMA,
        ],
        compiler_params=pltpu.CompilerParams(use_tc_tiling_on_sc=False),
    )(_sc_scatter_body)


def _sc_scatter_body(e_flat, recv2d, agg_out, idx_v, pay_v, zbuf, agg_sh, sem):
    cid = lax.axis_index("c")
    sid = lax.axis_index("s")
    base = sid * ZR

    def zrow(i2, carry):
        zbuf[i2, pl.ds(0, 16)] = jnp.zeros((16,), jnp.float32)
        return carry
    lax.fori_loop(0, ZB, zrow, 0)

    for c01 in range(NPC):
        chunk = cid * NPC + c01
        for k in range(ZR // ZB):
            pltpu.sync_copy(zbuf, agg_sh.at[pl.ds(base + k * ZB, ZB)])
        plsc.subcore_barrier()

        def blk(b, carry):
            r0 = sid * S_ROWS + b * S_BLK
            pltpu.sync_copy(recv2d.at[pl.ds(r0, S_BLK)], idx_v)
            pltpu.sync_copy(e_flat.at[pl.ds(chunk * EP + r0 * 128,
                                            S_BLK * 128)], pay_v)
            descs = [
                pltpu.async_copy(pay_v.at[pl.ds(j * 128, 128)],
                                 agg_sh.at[idx_v.at[j]], sem, add=True)
                for j in range(S_BLK)
            ]
            for d in descs:
                d.wait()
            return carry
        lax.fori_loop(0, S_NBLK, blk, 0)
        plsc.subcore_barrier()

        for k in range(ZR // ZB):
            pltpu.sync_copy(
                agg_sh.at[pl.ds(base + k * ZB, ZB)],
                agg_out.at[pl.ds(chunk * NP + base + k * ZB, ZB)])


def _edge_body(edges_ref, rf_ref, sf_ref, w1e, w1r, w1s, b1, w2, b2, w3, b3,
               e_out, se_out):
    i = pl.program_id(0)
    x = (edges_ref[...] @ w1e[...] + rf_ref[...] @ w1r[...]
         + sf_ref[...] @ w1s[...] + b1[...])
    h = _selu(x)
    h = _selu(h @ w2[...] + b2[...])
    h = _selu(h @ w3[...] + b3[...])
    rows = i * EBLK + lax.broadcasted_iota(jnp.int32, (EBLK, 1), 0)
    h = jnp.where(rows < E, h, 0.0)
    for c in range(CHUNKS):
        e_out[c] = h[:, c * CW:(c + 1) * CW]

    @pl.when(i == 0)
    def _():
        se_out[...] = jnp.zeros_like(se_out)
    se_out[...] += jnp.sum(h, axis=0, keepdims=True)


def _edge_call(edges_p, recvf, sendf, w1e, w1r, w1s, b1, w2, b2, w3, b3):
    full = lambda shape: pl.BlockSpec(shape, lambda i: (0, 0))
    return pl.pallas_call(
        _edge_body,
        grid=(NEB,),
        in_specs=[
            pl.BlockSpec((EBLK, 2), lambda i: (i, 0)),
            pl.BlockSpec((EBLK, 32), lambda i: (i, 0)),
            pl.BlockSpec((EBLK, 32), lambda i: (i, 0)),
            full((2, 128)), full((32, 128)), full((32, 128)), full((1, 128)),
            full((128, 256)), full((1, 256)), full((256, 128)), full((1, 128)),
        ],
        out_specs=[
            pl.BlockSpec((CHUNKS, EBLK, CW), lambda i: (0, i, 0)),
            pl.BlockSpec((1, 128), lambda i: (0, 0)),
        ],
        out_shape=[
            jax.ShapeDtypeStruct((CHUNKS, EP, CW), jnp.float32),
            jax.ShapeDtypeStruct((1, 128), jnp.float32),
        ],
        compiler_params=pltpu.CompilerParams(
            dimension_semantics=("arbitrary",)),
    )(edges_p, recvf, sendf, w1e, w1r, w1s, b1, w2, b2, w3, b3)


def _node_body(agg_ref, nodes_ref, wn1, bn1, wn2, bn2, se_ref,
               wg1a, wg1b, bg1, wg2, bg2, wg3, bg3, g_ref, sn_ref):
    i = pl.program_id(0)
    w = wn1[...]
    npre = nodes_ref[...] @ w[128:145, :] + bn1[...]
    for c in range(CHUNKS):
        npre += agg_ref[c] @ w[c * CW:(c + 1) * CW, :]
    n = _selu(npre) @ wn2[...] + bn2[...]

    @pl.when(i == 0)
    def _():
        sn_ref[...] = jnp.zeros_like(sn_ref)
    sn_ref[...] += jnp.sum(n, axis=0, keepdims=True)

    @pl.when(i == NNB - 1)
    def _():
        g1 = _selu(se_ref[...] @ wg1a[...] + sn_ref[...] @ wg1b[...]
                   + bg1[...])
        g2 = _selu(g1 @ wg2[...] + bg2[...])
        g_ref[...] = g2 @ wg3[...] + bg3[...]


def _node_call(agg, nodes, wn1, bn1, wn2p, bn2p, sum_e,
               wg1a, wg1b, bg1, wg2, bg2, wg3, bg3):
    full = lambda shape: pl.BlockSpec(shape, lambda i: (0, 0))
    return pl.pallas_call(
        _node_body,
        grid=(NNB,),
        in_specs=[
            pl.BlockSpec((CHUNKS, NB, CW), lambda i: (0, i, 0)),
            pl.BlockSpec((NB, 17), lambda i: (i, 0)),
            full((145, 100)), full((1, 100)), full((100, 128)), full((1, 128)),
            full((1, 128)),
            full((128, 100)), full((128, 100)), full((1, 100)),
            full((100, 50)), full((1, 50)), full((50, 7)), full((1, 7)),
        ],
        out_specs=pl.BlockSpec((1, 7), lambda i: (0, 0)),
        out_shape=jax.ShapeDtypeStruct((1, 7), jnp.float32),
        scratch_shapes=[pltpu.VMEM((1, 128), jnp.float32)],
        compiler_params=pltpu.CompilerParams(
            dimension_semantics=("arbitrary",)),
    )(agg, nodes, wn1, bn1, wn2p, bn2p, sum_e,
      wg1a, wg1b, bg1, wg2, bg2, wg3, bg3)


def kernel(nodes, edges, senders, receivers,
           We1, be1, We2, be2, We3, be3,
           Wn1, bn1, Wn2, bn2,
           Wg1, bg1, Wg2, bg2, Wg3, bg3):
    nodes_pad = jnp.pad(nodes, ((0, 0), (0, 15)))
    send2d = jnp.pad(senders, (0, EP - E)).reshape(IDX_ROWS, 128)
    recv2d = jnp.pad(receivers, (0, EP - E)).reshape(IDX_ROWS, 128)
    edges_p = jnp.pad(edges, ((0, EP - E), (0, 0)))

    sendf, recvf = _sc_gather_kernel()(nodes_pad, send2d, recv2d)

    w1e = We1[0:2]
    w1r = jnp.pad(We1[2:19], ((0, 15), (0, 0)))
    w1s = jnp.pad(We1[19:36], ((0, 15), (0, 0)))
    e4, sum_e = _edge_call(edges_p, recvf, sendf, w1e, w1r, w1s,
                           be1.reshape(1, 128), We2, be2.reshape(1, 256),
                           We3, be3.reshape(1, 128))

    agg = _sc_scatter_kernel()(e4.reshape(CHUNKS * EP, CW), recv2d)

    wn2p = jnp.pad(Wn2, ((0, 0), (0, 78)))
    bn2p = jnp.pad(bn2, (0, 78)).reshape(1, 128)
    wg1a = Wg1[:128]
    wg1b = jnp.pad(Wg1[128:178], ((0, 78), (0, 0)))
    g = _node_call(agg.reshape(CHUNKS, NP, CW), nodes, Wn1,
                   bn1.reshape(1, 100), wn2p, bn2p, sum_e,
                   wg1a, wg1b, bg1.reshape(1, 100),
                   Wg2, bg2.reshape(1, 50), Wg3, bg3.reshape(1, 7))
    return g.reshape(7)


# speedup vs baseline: 1.8898x; 1.7595x over previous
import functools

import jax
import jax.numpy as jnp
from jax import lax
from jax.experimental import pallas as pl
from jax.experimental.pallas import tpu as pltpu
from jax.experimental.pallas import tpu_sc as plsc

N = 50000
NP = 50048
E = 800000
EP = 819200
EBLK = 2048
NEB = EP // EBLK
IDX_ROWS = EP // 128

NC, NS = 2, 16
NW = NC * NS

G_ROWS = IDX_ROWS // NW
G_BLK = 8
G_NBLK = G_ROWS // G_BLK

S_ROWS = IDX_ROWS // NS
S_BLK = 8
S_NBLK = S_ROWS // S_BLK

CHUNKS = 8
CW = 16
NPC = CHUNKS // NC
ZR = NP // NS
ZB = 136

NB = 1000
NNB = N // NB

_ALPHA = 1.6732632423543772
_SCALE = 1.0507009873554805


def _selu(x):
    return _SCALE * jnp.where(x > 0, x, _ALPHA * (jnp.exp(x) - 1.0))


@functools.cache
def _sc_mesh():
    return plsc.VectorSubcoreMesh(
        core_axis_name="c", subcore_axis_name="s",
        num_cores=NC, num_subcores=NS)


@functools.cache
def _sc_gather_kernel():
    return functools.partial(
        pl.kernel,
        out_type=(jax.ShapeDtypeStruct((EP, 32), jnp.float32),
                  jax.ShapeDtypeStruct((EP, 32), jnp.float32)),
        mesh=_sc_mesh(),
        scratch_types=[
            pltpu.VMEM((G_BLK, 128), jnp.int32),
            pltpu.VMEM((G_BLK * 128, 32), jnp.float32),
            pltpu.SemaphoreType.DMA,
        ],
        compiler_params=pltpu.CompilerParams(use_tc_tiling_on_sc=False),
    )(_sc_gather_body)


def _sc_gather_body(table, send2d, recv2d, out_s, out_r, idx_v, rows_v, sem):
    wid = lax.axis_index("s") * NC + lax.axis_index("c")
    row0 = wid * G_ROWS

    def run(idx2d, out):
        def blk(b, carry):
            r0 = row0 + b * G_BLK
            pltpu.sync_copy(idx2d.at[pl.ds(r0, G_BLK)], idx_v)
            descs = [
                pltpu.async_copy(table.at[idx_v.at[j]],
                                 rows_v.at[pl.ds(j * 128, 128)], sem)
                for j in range(G_BLK)
            ]
            for d in descs:
                d.wait()
            pltpu.sync_copy(rows_v, out.at[pl.ds(r0 * 128, G_BLK * 128)])
            return carry
        lax.fori_loop(0, G_NBLK, blk, 0)

    run(send2d, out_s)
    run(recv2d, out_r)


@functools.cache
def _sc_scatter_kernel():
    return functools.partial(
        pl.kernel,
        out_type=jax.ShapeDtypeStruct((NP, 128), jnp.float32),
        mesh=_sc_mesh(),
        scratch_types=[
            pltpu.VMEM((S_BLK, 128), jnp.int32),
            pltpu.VMEM((S_BLK * 128, CW), jnp.float32),
            pltpu.VMEM((ZB, CW), jnp.float32),
            pltpu.VMEM_SHARED((NP, CW), jnp.float32),
            pltpu.SemaphoreType.DMA,
        ],
        compiler_params=pltpu.CompilerParams(use_tc_tiling_on_sc=False),
    )(_sc_scatter_body)


def _sc_scatter_body(e_hbm, recv2d, agg_out, idx_v, pay_v, zbuf, agg_sh, sem):
    cid = lax.axis_index("c")
    sid = lax.axis_index("s")
    base = sid * ZR

    def zrow(i2, carry):
        zbuf[i2, pl.ds(0, 16)] = jnp.zeros((16,), jnp.float32)
        return carry
    lax.fori_loop(0, ZB, zrow, 0)

    for c01 in range(NPC):
        chunk = cid * NPC + c01
        col = chunk * CW
        for k in range(ZR // ZB):
            pltpu.sync_copy(zbuf, agg_sh.at[pl.ds(base + k * ZB, ZB)])
        plsc.subcore_barrier()

        def blk(b, carry):
            r0 = sid * S_ROWS + b * S_BLK
            pltpu.sync_copy(recv2d.at[pl.ds(r0, S_BLK)], idx_v)
            pltpu.sync_copy(
                e_hbm.at[pl.ds(r0 * 128, S_BLK * 128), pl.ds(col, CW)],
                pay_v)
            descs = [
                pltpu.async_copy(pay_v.at[pl.ds(j * 128, 128)],
                                 agg_sh.at[idx_v.at[j]], sem, add=True)
                for j in range(S_BLK)
            ]
            for d in descs:
                d.wait()
            return carry
        lax.fori_loop(0, S_NBLK, blk, 0)
        plsc.subcore_barrier()

        for k in range(ZR // ZB):
            pltpu.sync_copy(
                agg_sh.at[pl.ds(base + k * ZB, ZB)],
                agg_out.at[pl.ds(base + k * ZB, ZB), pl.ds(col, CW)])


def _edge_body(edges_ref, rf_ref, sf_ref, w1e, w1r, w1s, b1, w2, b2, w3, b3,
               e_out, se_out):
    i = pl.program_id(0)
    x = (edges_ref[...] @ w1e[...] + rf_ref[...] @ w1r[...]
         + sf_ref[...] @ w1s[...] + b1[...])
    h = _selu(x)
    h = _selu(h @ w2[...] + b2[...])
    h = _selu(h @ w3[...] + b3[...])
    rows = i * EBLK + lax.broadcasted_iota(jnp.int32, (EBLK, 1), 0)
    h = jnp.where(rows < E, h, 0.0)
    e_out[...] = h

    @pl.when(i == 0)
    def _():
        se_out[...] = jnp.zeros_like(se_out)
    se_out[...] += jnp.sum(h, axis=0, keepdims=True)


def _edge_call(edges_p, recvf, sendf, w1e, w1r, w1s, b1, w2, b2, w3, b3):
    full = lambda shape: pl.BlockSpec(shape, lambda i: (0, 0))
    return pl.pallas_call(
        _edge_body,
        grid=(NEB,),
        in_specs=[
            pl.BlockSpec((EBLK, 2), lambda i: (i, 0)),
            pl.BlockSpec((EBLK, 32), lambda i: (i, 0)),
            pl.BlockSpec((EBLK, 32), lambda i: (i, 0)),
            full((2, 128)), full((32, 128)), full((32, 128)), full((1, 128)),
            full((128, 256)), full((1, 256)), full((256, 128)), full((1, 128)),
        ],
        out_specs=[
            pl.BlockSpec((EBLK, 128), lambda i: (i, 0)),
            pl.BlockSpec((1, 128), lambda i: (0, 0)),
        ],
        out_shape=[
            jax.ShapeDtypeStruct((EP, 128), jnp.float32),
            jax.ShapeDtypeStruct((1, 128), jnp.float32),
        ],
        compiler_params=pltpu.CompilerParams(
            dimension_semantics=("arbitrary",)),
    )(edges_p, recvf, sendf, w1e, w1r, w1s, b1, w2, b2, w3, b3)


def _node_body(agg_ref, nodes_ref, wn1, bn1, wn2, bn2, se_ref,
               wg1a, wg1b, bg1, wg2, bg2, wg3, bg3, g_ref, sn_ref):
    i = pl.program_id(0)
    w = wn1[...]
    npre = (agg_ref[...] @ w[0:128, :] + nodes_ref[...] @ w[128:145, :]
            + bn1[...])
    n = _selu(npre) @ wn2[...] + bn2[...]

    @pl.when(i == 0)
    def _():
        sn_ref[...] = jnp.zeros_like(sn_ref)
    sn_ref[...] += jnp.sum(n, axis=0, keepdims=True)

    @pl.when(i == NNB - 1)
    def _():
        g1 = _selu(se_ref[...] @ wg1a[...] + sn_ref[...] @ wg1b[...]
                   + bg1[...])
        g2 = _selu(g1 @ wg2[...] + bg2[...])
        g_ref[...] = g2 @ wg3[...] + bg3[...]


def _node_call(agg, nodes, wn1, bn1, wn2p, bn2p, sum_e,
               wg1a, wg1b, bg1, wg2, bg2, wg3, bg3):
    full = lambda shape: pl.BlockSpec(shape, lambda i: (0, 0))
    return pl.pallas_call(
        _node_body,
        grid=(NNB,),
        in_specs=[
            pl.BlockSpec((NB, 128), lambda i: (i, 0)),
            pl.BlockSpec((NB, 17), lambda i: (i, 0)),
            full((145, 100)), full((1, 100)), full((100, 128)), full((1, 128)),
            full((1, 128)),
            full((128, 100)), full((128, 100)), full((1, 100)),
            full((100, 50)), full((1, 50)), full((50, 7)), full((1, 7)),
        ],
        out_specs=pl.BlockSpec((1, 7), lambda i: (0, 0)),
        out_shape=jax.ShapeDtypeStruct((1, 7), jnp.float32),
        scratch_shapes=[pltpu.VMEM((1, 128), jnp.float32)],
        compiler_params=pltpu.CompilerParams(
            dimension_semantics=("arbitrary",)),
    )(agg, nodes, wn1, bn1, wn2p, bn2p, sum_e,
      wg1a, wg1b, bg1, wg2, bg2, wg3, bg3)


def kernel(nodes, edges, senders, receivers,
           We1, be1, We2, be2, We3, be3,
           Wn1, bn1, Wn2, bn2,
           Wg1, bg1, Wg2, bg2, Wg3, bg3):
    nodes_pad = jnp.pad(nodes, ((0, 0), (0, 15)))
    send2d = jnp.pad(senders, (0, EP - E)).reshape(IDX_ROWS, 128)
    recv2d = jnp.pad(receivers, (0, EP - E)).reshape(IDX_ROWS, 128)
    edges_p = jnp.pad(edges, ((0, EP - E), (0, 0)))

    sendf, recvf = _sc_gather_kernel()(nodes_pad, send2d, recv2d)

    w1e = We1[0:2]
    w1r = jnp.pad(We1[2:19], ((0, 15), (0, 0)))
    w1s = jnp.pad(We1[19:36], ((0, 15), (0, 0)))
    e_arr, sum_e = _edge_call(edges_p, recvf, sendf, w1e, w1r, w1s,
                              be1.reshape(1, 128), We2, be2.reshape(1, 256),
                              We3, be3.reshape(1, 128))

    agg = _sc_scatter_kernel()(e_arr, recv2d)

    wn2p = jnp.pad(Wn2, ((0, 0), (0, 78)))
    bn2p = jnp.pad(bn2, (0, 78)).reshape(1, 128)
    wg1a = Wg1[:128]
    wg1b = jnp.pad(Wg1[128:178], ((0, 78), (0, 0)))
    g = _node_call(agg, nodes, Wn1,
                   bn1.reshape(1, 100), wn2p, bn2p, sum_e,
                   wg1a, wg1b, bg1.reshape(1, 100),
                   Wg2, bg2.reshape(1, 50), Wg3, bg3.reshape(1, 7))
    return g.reshape(7)
